# baseline (speedup 1.0000x reference)
import functools

import jax
import jax.numpy as jnp
from jax.experimental import pallas as pl
from jax.experimental.pallas import tpu as pltpu

_TEMPERATURE = 0.5
_THRESHOLD = 0.5
_R = 256
_C = 2


def _fast_kernel(zi_ref, zjt_ref, cc_ref, cct_ref,
                 out_ref, pm_ref, cnt_ref, se_ref, pt_ref, *, n, r, nb):
    zjt = zjt_ref[...]
    cct = cct_ref[...]
    colid = jax.lax.broadcasted_iota(jnp.int32, (r, n), 1)
    rowid0 = jax.lax.broadcasted_iota(jnp.int32, (r, n), 0)

    def body(g, carry):
        r0 = g * r
        zi = zi_ref[pl.ds(r0, r), :]
        cc = cc_ref[pl.ds(r0, r), :]
        m2 = jnp.dot(cc, cct, preferred_element_type=jnp.float32)
        pos = (m2 > 2.0 * _THRESHOLD) | (colid == (rowid0 + r0))
        s = jnp.dot(zi, zjt, preferred_element_type=jnp.float32)
        cnt_ref[pl.ds(r0, r), :] = jnp.sum(pos.astype(jnp.int32), axis=1,
                                           keepdims=True)
        se_ref[pl.ds(r0, r), :] = jnp.sum(
            jnp.where(pos, 0.0, jnp.exp(2.0 * s)), axis=1, keepdims=True)
        pv = jnp.where(pos, s, -jnp.inf)
        mx1 = jnp.max(pv, axis=1, keepdims=True)
        eq = pv == mx1
        nmx = jnp.sum(eq.astype(jnp.float32), axis=1, keepdims=True)
        mx2c = jnp.max(jnp.where(eq, -jnp.inf, pv), axis=1, keepdims=True)
        mx2 = jnp.where(nmx >= 2.0, mx1, mx2c)
        pt_ref[pl.ds(r0, r), :] = jnp.concatenate([mx1, mx2], axis=1)
        return carry

    for g in range(nb):
        body(g, 0)

    cnt = cnt_ref[...]
    pmk = jnp.min(cnt, axis=0, keepdims=True)
    kidx = jax.lax.broadcasted_iota(jnp.int32, (n, _C), 1)
    log_neg = jnp.log(se_ref[...])
    keep = kidx < pmk
    plog = jnp.where(keep, 2.0 * pt_ref[...], 0.0)
    cell = (jnp.maximum(plog, log_neg)
            + jnp.log1p(jnp.exp(-jnp.abs(plog - log_neg))) - plog)
    total = jnp.sum(jnp.where(keep, cell, 0.0), axis=1, keepdims=True)
    total = jnp.sum(total, axis=0, keepdims=True)
    out_ref[...] = total / (jnp.float32(n) * pmk.astype(jnp.float32))
    pm_ref[...] = pmk


def _exact_kernel(zi_ref, zjt_ref, cc_ref, cct_ref,
                  out_ref, cnt_ref, se_ref, *, n, r, nb):
    zjt = zjt_ref[...]
    cct = cct_ref[...]
    colid = jax.lax.broadcasted_iota(jnp.int32, (r, n), 1)
    rowid0 = jax.lax.broadcasted_iota(jnp.int32, (r, n), 0)

    def masks_and_sim(g):
        r0 = g * r
        zi = zi_ref[pl.ds(r0, r), :]
        cc = cc_ref[pl.ds(r0, r), :]
        m2 = jnp.dot(cc, cct, preferred_element_type=jnp.float32)
        pos = (m2 > 2.0 * _THRESHOLD) | (colid == (rowid0 + r0))
        s = jnp.dot(zi, zjt, preferred_element_type=jnp.float32)
        return pos, s

    def phase_a(g, carry):
        r0 = g * r
        pos, s = masks_and_sim(g)
        cnt_ref[pl.ds(r0, r), :] = jnp.sum(pos.astype(jnp.int32), axis=1,
                                           keepdims=True)
        se_ref[pl.ds(r0, r), :] = jnp.sum(
            jnp.where(pos, 0.0, jnp.exp(2.0 * s)), axis=1, keepdims=True)
        return carry

    jax.lax.fori_loop(0, nb, phase_a, 0)

    cnt_all = cnt_ref[...]
    pm = jnp.min(cnt_all)
    mp = jnp.max(cnt_all)
    n_excl = mp - pm

    def phase_b(g, total):
        r0 = g * r
        pos, s = masks_and_sim(g)
        excl = mp - cnt_ref[pl.ds(r0, r), :]
        sumexp = se_ref[pl.ds(r0, r), :]

        def neg_body(k, c):
            nv, corr = c
            mn = jnp.min(nv, axis=1, keepdims=True)
            corr = corr + jnp.where(k < excl, jnp.exp(2.0 * mn), 0.0)
            first = jnp.min(jnp.where(nv == mn, colid, n), axis=1,
                            keepdims=True)
            nv = jnp.where(colid == first, jnp.inf, nv)
            return nv, corr

        nv0 = jnp.where(pos, jnp.inf, s)
        corr0 = jnp.zeros((r, 1), jnp.float32)
        _, corr = jax.lax.fori_loop(0, n_excl, neg_body, (nv0, corr0))
        log_neg = jnp.log(sumexp - corr)

        def pos_body(k, c):
            pv, acc = c
            mx = jnp.max(pv, axis=1, keepdims=True)
            plog = 2.0 * mx
            cell = (jnp.maximum(plog, log_neg)
                    + jnp.log1p(jnp.exp(-jnp.abs(plog - log_neg))) - plog)
            acc = acc + cell
            first = jnp.min(jnp.where(pv == mx, colid, n), axis=1,
                            keepdims=True)
            pv = jnp.where(colid == first, -jnp.inf, pv)
            return pv, acc

        pv0 = jnp.where(pos, s, -jnp.inf)
        acc0 = jnp.zeros((r, 1), jnp.float32)
        _, acc = jax.lax.fori_loop(0, pm, pos_body, (pv0, acc0))
        return total + jnp.sum(acc, axis=0, keepdims=True)

    total = jax.lax.fori_loop(0, nb, phase_b,
                              jnp.zeros((1, 1), jnp.float32))
    out_ref[...] = total / (jnp.float32(n) * pm.astype(jnp.float32))


def kernel(z_i, z_j, c_i, c_j):
    n = z_i.shape[0]
    r = _R if n % _R == 0 else n
    nb = n // r
    cc = jnp.concatenate([c_i, c_j], axis=1)
    args = (z_i, z_j.T, cc, cc.T)
    loss_f, pm = pl.pallas_call(
        functools.partial(_fast_kernel, n=n, r=r, nb=nb),
        out_shape=(jax.ShapeDtypeStruct((1, 1), jnp.float32),
                   jax.ShapeDtypeStruct((1, 1), jnp.int32)),
        scratch_shapes=[
            pltpu.VMEM((n, 1), jnp.int32),
            pltpu.VMEM((n, 1), jnp.float32),
            pltpu.VMEM((n, _C), jnp.float32),
        ],
    )(*args)

    def take_fast(_):
        return loss_f[0, 0]

    def exact(_):
        out = pl.pallas_call(
            functools.partial(_exact_kernel, n=n, r=r, nb=nb),
            out_shape=jax.ShapeDtypeStruct((1, 1), jnp.float32),
            scratch_shapes=[
                pltpu.VMEM((n, 1), jnp.int32),
                pltpu.VMEM((n, 1), jnp.float32),
            ],
        )(*args)
        return out[0, 0]

    return jax.lax.cond(pm[0, 0] <= _C, take_fast, exact, 0)


# speedup vs baseline: 1.0490x; 1.0490x over previous
import functools

import jax
import jax.numpy as jnp
from jax.experimental import pallas as pl
from jax.experimental.pallas import tpu as pltpu

_TEMPERATURE = 0.5
_THRESHOLD = 0.5
_R = 256
_C = 2


def _fast_kernel(zi_ref, zjt_ref, ci_ref, cit_ref, cj_ref, cjt_ref,
                 out_ref, pm_ref, cnt_ref, se_ref, pt_ref, *, n, r, nb):
    zjt = zjt_ref[...]
    cit = cit_ref[...]
    cjt = cjt_ref[...]
    colid = jax.lax.broadcasted_iota(jnp.int32, (r, n), 1)
    rowid0 = jax.lax.broadcasted_iota(jnp.int32, (r, n), 0)

    def body(g, carry):
        r0 = g * r
        zi = zi_ref[pl.ds(r0, r), :]
        ci = ci_ref[pl.ds(r0, r), :]
        cj = cj_ref[pl.ds(r0, r), :]
        m = 0.5 * (jnp.dot(ci, cit, preferred_element_type=jnp.float32)
                   + jnp.dot(cj, cjt, preferred_element_type=jnp.float32))
        is_diag = colid == (rowid0 + r0)
        m = jnp.where(is_diag, 1.0, m)
        pos = m > _THRESHOLD
        s = jnp.dot(zi, zjt, preferred_element_type=jnp.float32)
        cnt_ref[pl.ds(r0, r), :] = jnp.sum(pos.astype(jnp.int32), axis=1,
                                           keepdims=True)
        se_ref[pl.ds(r0, r), :] = jnp.sum(
            jnp.where(pos, 0.0, jnp.exp(2.0 * s)), axis=1, keepdims=True)
        pv = jnp.where(pos, s, -jnp.inf)
        mx1 = jnp.max(pv, axis=1, keepdims=True)
        eq = pv == mx1
        nmx = jnp.sum(eq.astype(jnp.float32), axis=1, keepdims=True)
        mx2c = jnp.max(jnp.where(eq, -jnp.inf, pv), axis=1, keepdims=True)
        mx2 = jnp.where(nmx >= 2.0, mx1, mx2c)
        pt_ref[pl.ds(r0, r), :] = jnp.concatenate([mx1, mx2], axis=1)
        return carry

    for g in range(nb):
        body(g, 0)

    cnt = cnt_ref[...]
    pmk = jnp.min(cnt, axis=0, keepdims=True)
    kidx = jax.lax.broadcasted_iota(jnp.int32, (n, _C), 1)
    log_neg = jnp.log(se_ref[...])
    keep = kidx < pmk
    plog = jnp.where(keep, 2.0 * pt_ref[...], 0.0)
    cell = (jnp.maximum(plog, log_neg)
            + jnp.log1p(jnp.exp(-jnp.abs(plog - log_neg))) - plog)
    total = jnp.sum(jnp.where(keep, cell, 0.0), axis=1, keepdims=True)
    total = jnp.sum(total, axis=0, keepdims=True)
    out_ref[...] = total / (jnp.float32(n) * pmk.astype(jnp.float32))
    pm_ref[...] = pmk


def _exact_kernel(zi_ref, zjt_ref, ci_ref, cit_ref, cj_ref, cjt_ref,
                  out_ref, cnt_ref, se_ref, *, n, r, nb):
    zjt = zjt_ref[...]
    cit = cit_ref[...]
    cjt = cjt_ref[...]
    colid = jax.lax.broadcasted_iota(jnp.int32, (r, n), 1)
    rowid0 = jax.lax.broadcasted_iota(jnp.int32, (r, n), 0)

    def masks_and_sim(g):
        r0 = g * r
        zi = zi_ref[pl.ds(r0, r), :]
        ci = ci_ref[pl.ds(r0, r), :]
        cj = cj_ref[pl.ds(r0, r), :]
        m = 0.5 * (jnp.dot(ci, cit, preferred_element_type=jnp.float32)
                   + jnp.dot(cj, cjt, preferred_element_type=jnp.float32))
        is_diag = colid == (rowid0 + r0)
        m = jnp.where(is_diag, 1.0, m)
        pos = m > _THRESHOLD
        s = jnp.dot(zi, zjt, preferred_element_type=jnp.float32)
        return pos, s

    def phase_a(g, carry):
        r0 = g * r
        pos, s = masks_and_sim(g)
        cnt_ref[pl.ds(r0, r), :] = jnp.sum(pos.astype(jnp.int32), axis=1,
                                           keepdims=True)
        se_ref[pl.ds(r0, r), :] = jnp.sum(
            jnp.where(pos, 0.0, jnp.exp(2.0 * s)), axis=1, keepdims=True)
        return carry

    jax.lax.fori_loop(0, nb, phase_a, 0)

    cnt_all = cnt_ref[...]
    pm = jnp.min(cnt_all)
    mp = jnp.max(cnt_all)
    n_excl = mp - pm

    def phase_b(g, total):
        r0 = g * r
        pos, s = masks_and_sim(g)
        excl = mp - cnt_ref[pl.ds(r0, r), :]
        sumexp = se_ref[pl.ds(r0, r), :]

        def neg_body(k, c):
            nv, corr = c
            mn = jnp.min(nv, axis=1, keepdims=True)
            corr = corr + jnp.where(k < excl, jnp.exp(2.0 * mn), 0.0)
            first = jnp.min(jnp.where(nv == mn, colid, n), axis=1,
                            keepdims=True)
            nv = jnp.where(colid == first, jnp.inf, nv)
            return nv, corr

        nv0 = jnp.where(pos, jnp.inf, s)
        corr0 = jnp.zeros((r, 1), jnp.float32)
        _, corr = jax.lax.fori_loop(0, n_excl, neg_body, (nv0, corr0))
        log_neg = jnp.log(sumexp - corr)

        def pos_body(k, c):
            pv, acc = c
            mx = jnp.max(pv, axis=1, keepdims=True)
            plog = 2.0 * mx
            cell = (jnp.maximum(plog, log_neg)
                    + jnp.log1p(jnp.exp(-jnp.abs(plog - log_neg))) - plog)
            acc = acc + cell
            first = jnp.min(jnp.where(pv == mx, colid, n), axis=1,
                            keepdims=True)
            pv = jnp.where(colid == first, -jnp.inf, pv)
            return pv, acc

        pv0 = jnp.where(pos, s, -jnp.inf)
        acc0 = jnp.zeros((r, 1), jnp.float32)
        _, acc = jax.lax.fori_loop(0, pm, pos_body, (pv0, acc0))
        return total + jnp.sum(acc, axis=0, keepdims=True)

    total = jax.lax.fori_loop(0, nb, phase_b,
                              jnp.zeros((1, 1), jnp.float32))
    out_ref[...] = total / (jnp.float32(n) * pm.astype(jnp.float32))


def kernel(z_i, z_j, c_i, c_j):
    n = z_i.shape[0]
    r = _R if n % _R == 0 else n
    nb = n // r
    args = (z_i, z_j.T, c_i, c_i.T, c_j, c_j.T)
    loss_f, pm = pl.pallas_call(
        functools.partial(_fast_kernel, n=n, r=r, nb=nb),
        out_shape=(jax.ShapeDtypeStruct((1, 1), jnp.float32),
                   jax.ShapeDtypeStruct((1, 1), jnp.int32)),
        scratch_shapes=[
            pltpu.VMEM((n, 1), jnp.int32),
            pltpu.VMEM((n, 1), jnp.float32),
            pltpu.VMEM((n, _C), jnp.float32),
        ],
    )(*args)

    def take_fast(_):
        return loss_f[0, 0]

    def exact(_):
        out = pl.pallas_call(
            functools.partial(_exact_kernel, n=n, r=r, nb=nb),
            out_shape=jax.ShapeDtypeStruct((1, 1), jnp.float32),
            scratch_shapes=[
                pltpu.VMEM((n, 1), jnp.int32),
                pltpu.VMEM((n, 1), jnp.float32),
            ],
        )(*args)
        return out[0, 0]

    return jax.lax.cond(pm[0, 0] <= _C, take_fast, exact, 0)
